# baseline (speedup 1.0000x reference)
import functools

import jax
import jax.numpy as jnp
from jax import lax
from jax.experimental import pallas as pl
from jax.experimental.pallas import tpu as pltpu
from jax.experimental.pallas import tpu_sc as plsc

T = 2048
TPAD = 2056
D = 1024
H = 2048
NE = 8
CAP = 768
SLOTS = NE * CAP
GPAD = 6208
SPW = SLOTS // 32
GCH = 64
TPW = T // 32
BH = 1024


def _route_body(x_ref, g_ref, gw_ref, xn_ref, da1_ref, da2_ref,
                ga1_ref, ga2_ref, cw1_ref, cw2_ref):
    x = x_ref[...]
    ssq = jnp.sum(x * x, axis=1, keepdims=True)
    xn = x * lax.rsqrt(ssq / D + 1e-6) * g_ref[...]
    xn_ref[0:T, :] = xn
    xn_ref[T:TPAD, :] = jnp.zeros((TPAD - T, D), jnp.float32)
    logits = lax.dot_general(xn, gw_ref[...], (((1,), (1,)), ((), ())),
                             preferred_element_type=jnp.float32)
    m = jnp.max(logits, axis=1, keepdims=True)
    ex = jnp.exp(logits - m)
    probs = ex / jnp.sum(ex, axis=1, keepdims=True)
    iota = lax.broadcasted_iota(jnp.int32, (T, NE), 1)
    p1 = jnp.max(probs, axis=1, keepdims=True)
    e1 = jnp.min(jnp.where(probs == p1, iota, NE), axis=1, keepdims=True)
    probs2 = jnp.where(iota == e1, -1.0, probs)
    p2 = jnp.max(probs2, axis=1, keepdims=True)
    e2 = jnp.min(jnp.where(probs2 == p2, iota, NE), axis=1, keepdims=True)
    denom = p1 + p2 + 1e-10
    w1v = p1 / denom
    w2v = p2 / denom
    oh1 = (iota == e1).astype(jnp.int32)
    oh2 = (iota == e2).astype(jnp.int32)

    def cumsum0(a):
        s = 1
        while s < T:
            a = a + jnp.concatenate(
                [jnp.zeros((s, NE), jnp.int32), a[:T - s]], axis=0)
            s *= 2
        return a

    pos1 = jnp.sum(cumsum0(oh1) * oh1, axis=1, keepdims=True) - 1
    pos2 = jnp.sum(cumsum0(oh2) * oh2, axis=1, keepdims=True) - 1
    ok1 = pos1 < CAP
    ok2 = pos2 < CAP
    da1_ref[...] = jnp.where(ok1, e1 * CAP + pos1, SLOTS)
    da2_ref[...] = jnp.where(ok2, e2 * CAP + pos2, SLOTS)
    ga1_ref[...] = jnp.where(ok1, e1 * CAP + pos1, 0)
    ga2_ref[...] = jnp.where(ok2, e2 * CAP + pos2, 0)
    cw1_ref[...] = jnp.where(ok1, w1v, 0.0)
    cw2_ref[...] = jnp.where(ok2, w2v, 0.0)


_route = pl.pallas_call(
    _route_body,
    out_shape=[
        jax.ShapeDtypeStruct((TPAD, D), jnp.float32),
        jax.ShapeDtypeStruct((T, 1), jnp.int32),
        jax.ShapeDtypeStruct((T, 1), jnp.int32),
        jax.ShapeDtypeStruct((T, 1), jnp.int32),
        jax.ShapeDtypeStruct((T, 1), jnp.int32),
        jax.ShapeDtypeStruct((T, 1), jnp.float32),
        jax.ShapeDtypeStruct((T, 1), jnp.float32),
    ],
)


@functools.cache
def _make_dispatch():
    mesh = plsc.VectorSubcoreMesh(core_axis_name="c", subcore_axis_name="s")

    @functools.partial(
        pl.kernel,
        mesh=mesh,
        compiler_params=pltpu.CompilerParams(needs_layout_passes=False),
        out_type=jax.ShapeDtypeStruct((2 * SLOTS, D), jnp.float32),
        scratch_types=[
            pltpu.VMEM((T,), jnp.int32),
            pltpu.VMEM((T,), jnp.int32),
            pltpu.VMEM((GPAD,), jnp.int32),
            pltpu.VMEM((GPAD,), jnp.int32),
            pltpu.VMEM((GCH,), jnp.int32),
            pltpu.VMEM((GCH, D), jnp.float32),
            pltpu.SemaphoreType.DMA,
        ],
    )
    def _dispatch(xn_hbm, da1_hbm, da2_hbm, ei_hbm,
                  da1_v, da2_v, gl1_v, gl2_v, idx_v, rows_v, sem):
        c = lax.axis_index("c")
        s = lax.axis_index("s")
        wid = s * 2 + c
        pltpu.sync_copy(da1_hbm, da1_v)
        pltpu.sync_copy(da2_hbm, da2_v)
        sentinel = jnp.full((16,), T, jnp.int32)

        def init_body(i, _):
            gl1_v[pl.ds(i * 16, 16)] = sentinel
            gl2_v[pl.ds(i * 16, 16)] = sentinel
            return 0

        lax.fori_loop(0, GPAD // 16, init_body, 0)

        lane = lax.broadcasted_iota(jnp.int32, (16,), 0)

        def scat_body(i, _):
            tok = lane + i * 16
            plsc.store_scatter(gl1_v, [da1_v[pl.ds(i * 16, 16)]], tok)
            plsc.store_scatter(gl2_v, [da2_v[pl.ds(i * 16, 16)]], tok)
            return 0

        lax.fori_loop(0, T // 16, scat_body, 0)

        base = wid * SPW
        for st, gl_v in ((0, gl1_v), (1, gl2_v)):
            for ch in range(SPW // GCH):
                for k in range(GCH // 16):
                    idx_v[pl.ds(k * 16, 16)] = gl_v[
                        pl.ds(base + ch * GCH + k * 16, 16)]
                pltpu.async_copy(xn_hbm.at[idx_v], rows_v, sem).wait()
                pltpu.sync_copy(
                    rows_v,
                    ei_hbm.at[pl.ds(st * SLOTS + base + ch * GCH, GCH)])

    return _dispatch


def _ffn_body(ei_ref, w1_ref, w2_ref, w3_ref, out_ref):
    hb = pl.program_id(1)
    x = ei_ref[0] + ei_ref[1]
    h1 = lax.dot_general(x, w1_ref[0], (((1,), (0,)), ((), ())),
                         preferred_element_type=jnp.float32)
    h2 = lax.dot_general(x, w2_ref[0], (((1,), (0,)), ((), ())),
                         preferred_element_type=jnp.float32)
    h = h1 * (1.0 / (1.0 + jnp.exp(-h1))) * h2
    part = lax.dot_general(h, w3_ref[0], (((1,), (0,)), ((), ())),
                           preferred_element_type=jnp.float32)

    @pl.when(hb == 0)
    def _():
        out_ref[...] = part

    @pl.when(hb != 0)
    def _():
        out_ref[...] += part


_ffn = pl.pallas_call(
    _ffn_body,
    grid=(NE, H // BH),
    in_specs=[
        pl.BlockSpec((2, CAP, D), lambda e, hb: (0, e, 0)),
        pl.BlockSpec((1, D, BH), lambda e, hb: (e, 0, hb)),
        pl.BlockSpec((1, D, BH), lambda e, hb: (e, 0, hb)),
        pl.BlockSpec((1, BH, D), lambda e, hb: (e, hb, 0)),
    ],
    out_specs=pl.BlockSpec((CAP, D), lambda e, hb: (e, 0)),
    out_shape=jax.ShapeDtypeStruct((SLOTS, D), jnp.float32),
)


@functools.cache
def _make_combine_gather():
    mesh = plsc.VectorSubcoreMesh(core_axis_name="c", subcore_axis_name="s")

    @functools.partial(
        pl.kernel,
        mesh=mesh,
        out_type=jax.ShapeDtypeStruct((2 * T, D), jnp.float32),
        scratch_types=[
            pltpu.VMEM((TPW,), jnp.int32),
            pltpu.VMEM((TPW, D), jnp.float32),
            pltpu.SemaphoreType.DMA,
        ],
    )
    def _combine_gather(eo_hbm, ga1_hbm, ga2_hbm, g_hbm, idx_v, rows_v, sem):
        c = lax.axis_index("c")
        s = lax.axis_index("s")
        wid = s * 2 + c
        base = wid * TPW
        pltpu.sync_copy(ga1_hbm.at[pl.ds(base, TPW)], idx_v)
        pltpu.async_copy(eo_hbm.at[idx_v], rows_v, sem).wait()
        pltpu.sync_copy(rows_v, g_hbm.at[pl.ds(base, TPW)])
        pltpu.sync_copy(ga2_hbm.at[pl.ds(base, TPW)], idx_v)
        pltpu.async_copy(eo_hbm.at[idx_v], rows_v, sem).wait()
        pltpu.sync_copy(rows_v, g_hbm.at[pl.ds(T + base, TPW)])

    return _combine_gather


def _wadd_body(g_ref, cw1_ref, cw2_ref, out_ref):
    out_ref[...] = cw1_ref[...] * g_ref[0] + cw2_ref[...] * g_ref[1]


_wadd = pl.pallas_call(
    _wadd_body,
    out_shape=jax.ShapeDtypeStruct((T, D), jnp.float32),
)


def kernel(x, norm_g, gate_w, w1, w2, w3):
    b, s, d = x.shape
    xf = x.reshape(T, D)
    xn, da1, da2, ga1, ga2, cw1, cw2 = _route(xf, norm_g.reshape(1, D), gate_w)
    ei = _make_dispatch()(xn, da1.reshape(T), da2.reshape(T))
    eo = _ffn(ei.reshape(2, SLOTS, D), w1, w2, w3)
    g = _make_combine_gather()(eo, ga1.reshape(T), ga2.reshape(T))
    out = _wadd(g.reshape(2, T, D), cw1, cw2)
    return out.reshape(b, s, d)


# speedup vs baseline: 2.7885x; 2.7885x over previous
import functools

import jax
import jax.numpy as jnp
from jax import lax
from jax.experimental import pallas as pl
from jax.experimental.pallas import tpu as pltpu
from jax.experimental.pallas import tpu_sc as plsc

T = 2048
D = 1024
H = 2048
NE = 8
CAP = 768
SLOTS = NE * CAP
DC = 128
NCH = D // DC
BUFROWS = 6272
ZR = BUFROWS // 16
ORR = SLOTS // 16
TPT = T // 16
TPW = T // 32
BH = 1024


def _route_body(x_ref, g_ref, gw_ref, xn_ref, da1_ref, da2_ref,
                ga1_ref, ga2_ref, cw1_ref, cw2_ref):
    x = x_ref[...]
    ssq = jnp.sum(x * x, axis=1, keepdims=True)
    xn = x * lax.rsqrt(ssq / D + 1e-6) * g_ref[...]
    xn_ref[...] = xn
    logits = lax.dot_general(gw_ref[...], xn, (((1,), (1,)), ((), ())),
                             preferred_element_type=jnp.float32)
    m = jnp.max(logits, axis=0, keepdims=True)
    ex = jnp.exp(logits - m)
    probs = ex / jnp.sum(ex, axis=0, keepdims=True)
    iota = lax.broadcasted_iota(jnp.int32, (NE, T), 0)
    p1 = jnp.max(probs, axis=0, keepdims=True)
    e1 = jnp.min(jnp.where(probs == p1, iota, NE), axis=0, keepdims=True)
    probs2 = jnp.where(iota == e1, -1.0, probs)
    p2 = jnp.max(probs2, axis=0, keepdims=True)
    e2 = jnp.min(jnp.where(probs2 == p2, iota, NE), axis=0, keepdims=True)
    denom = p1 + p2 + 1e-10
    w1v = p1 / denom
    w2v = p2 / denom
    oh1 = (iota == e1).astype(jnp.float32)
    oh2 = (iota == e2).astype(jnp.float32)

    NB, BSZ = 16, T // 16
    bi = lax.broadcasted_iota(jnp.int32, (BSZ, BSZ), 0)
    bj = lax.broadcasted_iota(jnp.int32, (BSZ, BSZ), 1)
    triu = (bi <= bj).astype(jnp.float32)

    def cumsum0(a):
        ab = a.reshape(NE, NB, BSZ)
        cin = lax.dot_general(ab, triu, (((2,), (0,)), ((), ())))
        tot = cin[:, :, BSZ - 1]
        s = 1
        while s < NB:
            tot_sh = jnp.concatenate(
                [jnp.zeros((NE, s), jnp.float32), tot[:, :NB - s]], axis=1)
            tot = tot + tot_sh
            s *= 2
        pref = jnp.concatenate(
            [jnp.zeros((NE, 1), jnp.float32), tot[:, :NB - 1]], axis=1)
        return (cin + pref[:, :, None]).reshape(NE, T)

    pos1 = jnp.sum(cumsum0(oh1) * oh1, axis=0, keepdims=True).astype(jnp.int32) - 1
    pos2 = jnp.sum(cumsum0(oh2) * oh2, axis=0, keepdims=True).astype(jnp.int32) - 1
    ok1 = pos1 < CAP
    ok2 = pos2 < CAP
    da1_ref[...] = jnp.where(ok1, e1 * CAP + pos1, SLOTS)
    da2_ref[...] = jnp.where(ok2, e2 * CAP + pos2, SLOTS)
    ga1_ref[...] = jnp.where(ok1, e1 * CAP + pos1, 0)
    ga2_ref[...] = jnp.where(ok2, e2 * CAP + pos2, 0)
    cw1_ref[...] = jnp.where(ok1, w1v, 0.0)
    cw2_ref[...] = jnp.where(ok2, w2v, 0.0)


_route = pl.pallas_call(
    _route_body,
    out_shape=[
        jax.ShapeDtypeStruct((T, D), jnp.float32),
        jax.ShapeDtypeStruct((1, T), jnp.int32),
        jax.ShapeDtypeStruct((1, T), jnp.int32),
        jax.ShapeDtypeStruct((1, T), jnp.int32),
        jax.ShapeDtypeStruct((1, T), jnp.int32),
        jax.ShapeDtypeStruct((1, T), jnp.float32),
        jax.ShapeDtypeStruct((1, T), jnp.float32),
    ],
)


@functools.cache
def _make_dispatch():
    mesh = plsc.VectorSubcoreMesh(core_axis_name="c", subcore_axis_name="s")

    @functools.partial(
        pl.kernel,
        mesh=mesh,
        out_type=jax.ShapeDtypeStruct((SLOTS, D), jnp.float32),
        scratch_types=[
            pltpu.VMEM((TPT, DC), jnp.float32),
            pltpu.VMEM((TPT,), jnp.int32),
            pltpu.VMEM((TPT,), jnp.int32),
            pltpu.VMEM_SHARED((BUFROWS, DC), jnp.float32),
            pltpu.SemaphoreType.DMA,
            pltpu.SemaphoreType.DMA,
        ],
    )
    def _dispatch(xn_hbm, da1_hbm, da2_hbm, zeros_hbm, ei_hbm,
                  rows_v, idx1_v, idx2_v, shared, sem0, sem1):
        c = lax.axis_index("c")
        s = lax.axis_index("s")
        base_t = s * TPT
        pltpu.sync_copy(da1_hbm.at[pl.ds(base_t, TPT)], idx1_v)
        pltpu.sync_copy(da2_hbm.at[pl.ds(base_t, TPT)], idx2_v)
        for cj in range(NCH // 2):
            j = c * (NCH // 2) + cj
            cp_z = pltpu.async_copy(zeros_hbm,
                                    shared.at[pl.ds(s * ZR, ZR)], sem0)
            cp_l = pltpu.async_copy(
                xn_hbm.at[pl.ds(base_t, TPT), pl.ds(j * DC, DC)],
                rows_v, sem1)
            cp_z.wait()
            cp_l.wait()
            plsc.subcore_barrier()
            c1 = pltpu.async_copy(rows_v, shared.at[idx1_v], sem0, add=True)
            c2 = pltpu.async_copy(rows_v, shared.at[idx2_v], sem1, add=True)
            c1.wait()
            c2.wait()
            plsc.subcore_barrier()
            pltpu.sync_copy(
                shared.at[pl.ds(s * ORR, ORR)],
                ei_hbm.at[pl.ds(s * ORR, ORR), pl.ds(j * DC, DC)])
            plsc.subcore_barrier()

    return _dispatch


def _ffn_body(ei_ref, w1_ref, w2_ref, w3_ref, out_ref):
    hb = pl.program_id(1)
    x = ei_ref[...]
    h1 = lax.dot_general(x, w1_ref[0], (((1,), (0,)), ((), ())),
                         preferred_element_type=jnp.float32)
    h2 = lax.dot_general(x, w2_ref[0], (((1,), (0,)), ((), ())),
                         preferred_element_type=jnp.float32)
    h = h1 * (1.0 / (1.0 + jnp.exp(-h1))) * h2
    part = lax.dot_general(h, w3_ref[0], (((1,), (0,)), ((), ())),
                           preferred_element_type=jnp.float32)

    @pl.when(hb == 0)
    def _():
        out_ref[...] = part

    @pl.when(hb != 0)
    def _():
        out_ref[...] += part


_ffn = pl.pallas_call(
    _ffn_body,
    grid=(NE, H // BH),
    in_specs=[
        pl.BlockSpec((CAP, D), lambda e, hb: (e, 0)),
        pl.BlockSpec((1, D, BH), lambda e, hb: (e, 0, hb)),
        pl.BlockSpec((1, D, BH), lambda e, hb: (e, 0, hb)),
        pl.BlockSpec((1, BH, D), lambda e, hb: (e, hb, 0)),
    ],
    out_specs=pl.BlockSpec((CAP, D), lambda e, hb: (e, 0)),
    out_shape=jax.ShapeDtypeStruct((SLOTS, D), jnp.float32),
)


@functools.cache
def _make_combine_gather():
    mesh = plsc.VectorSubcoreMesh(core_axis_name="c", subcore_axis_name="s")

    @functools.partial(
        pl.kernel,
        mesh=mesh,
        out_type=jax.ShapeDtypeStruct((2 * T, D), jnp.float32),
        scratch_types=[
            pltpu.VMEM((TPW,), jnp.int32),
            pltpu.VMEM((TPW, D), jnp.float32),
            pltpu.SemaphoreType.DMA,
        ],
    )
    def _combine_gather(eo_hbm, ga1_hbm, ga2_hbm, g_hbm, idx_v, rows_v, sem):
        c = lax.axis_index("c")
        s = lax.axis_index("s")
        wid = s * 2 + c
        base = wid * TPW
        pltpu.sync_copy(ga1_hbm.at[pl.ds(base, TPW)], idx_v)
        pltpu.async_copy(eo_hbm.at[idx_v], rows_v, sem).wait()
        pltpu.sync_copy(rows_v, g_hbm.at[pl.ds(base, TPW)])
        pltpu.sync_copy(ga2_hbm.at[pl.ds(base, TPW)], idx_v)
        pltpu.async_copy(eo_hbm.at[idx_v], rows_v, sem).wait()
        pltpu.sync_copy(rows_v, g_hbm.at[pl.ds(T + base, TPW)])

    return _combine_gather


def _wadd_body(g_ref, cw1_ref, cw2_ref, out_ref):
    out_ref[...] = cw1_ref[...] * g_ref[0] + cw2_ref[...] * g_ref[1]


_wadd = pl.pallas_call(
    _wadd_body,
    out_shape=jax.ShapeDtypeStruct((T, D), jnp.float32),
)


def kernel(x, norm_g, gate_w, w1, w2, w3):
    b, s, d = x.shape
    xf = x.reshape(T, D)
    xn, da1, da2, ga1, ga2, cw1, cw2 = _route(xf, norm_g.reshape(1, D), gate_w)
    zeros = jnp.zeros((ZR, DC), jnp.float32)
    ei = _make_dispatch()(xn, da1.reshape(T), da2.reshape(T), zeros)
    eo = _ffn(ei, w1, w2, w3)
    g = _make_combine_gather()(eo, ga1.reshape(T), ga2.reshape(T))
    out = _wadd(g.reshape(2, T, D), cw1.reshape(T, 1), cw2.reshape(T, 1))
    return out.reshape(b, s, d)


# speedup vs baseline: 2.9346x; 1.0524x over previous
import functools

import jax
import jax.numpy as jnp
from jax import lax
from jax.experimental import pallas as pl
from jax.experimental.pallas import tpu as pltpu
from jax.experimental.pallas import tpu_sc as plsc

T = 2048
D = 1024
H = 2048
NE = 8
CAP = 768
SLOTS = NE * CAP
DC = 128
NCH = D // DC
BUFROWS = 6400
ORW = SLOTS // 16
TPT = T // 16
TPW = T // 32
BH = 1024


def _route_body(x_ref, g_ref, gw_ref, xn_ref, da1_ref, da2_ref,
                ga1_ref, ga2_ref, cw1_ref, cw2_ref):
    x = x_ref[...]
    ssq = jnp.sum(x * x, axis=1, keepdims=True)
    xn = x * lax.rsqrt(ssq / D + 1e-6) * g_ref[...]
    xn_ref[...] = xn
    logits = lax.dot_general(gw_ref[...], xn, (((1,), (1,)), ((), ())),
                             preferred_element_type=jnp.float32)
    m = jnp.max(logits, axis=0, keepdims=True)
    ex = jnp.exp(logits - m)
    probs = ex / jnp.sum(ex, axis=0, keepdims=True)
    iota = lax.broadcasted_iota(jnp.int32, (NE, T), 0)
    p1 = jnp.max(probs, axis=0, keepdims=True)
    e1 = jnp.min(jnp.where(probs == p1, iota, NE), axis=0, keepdims=True)
    probs2 = jnp.where(iota == e1, -1.0, probs)
    p2 = jnp.max(probs2, axis=0, keepdims=True)
    e2 = jnp.min(jnp.where(probs2 == p2, iota, NE), axis=0, keepdims=True)
    denom = p1 + p2 + 1e-10
    w1v = p1 / denom
    w2v = p2 / denom
    oh1 = (iota == e1).astype(jnp.float32)
    oh2 = (iota == e2).astype(jnp.float32)

    NB, BSZ = 16, T // 16
    bi = lax.broadcasted_iota(jnp.int32, (BSZ, BSZ), 0)
    bj = lax.broadcasted_iota(jnp.int32, (BSZ, BSZ), 1)
    triu = (bi <= bj).astype(jnp.float32)

    def cumsum0(a):
        ab = a.reshape(NE, NB, BSZ)
        cin = lax.dot_general(ab, triu, (((2,), (0,)), ((), ())))
        tot = cin[:, :, BSZ - 1]
        s = 1
        while s < NB:
            tot_sh = jnp.concatenate(
                [jnp.zeros((NE, s), jnp.float32), tot[:, :NB - s]], axis=1)
            tot = tot + tot_sh
            s *= 2
        pref = jnp.concatenate(
            [jnp.zeros((NE, 1), jnp.float32), tot[:, :NB - 1]], axis=1)
        return (cin + pref[:, :, None]).reshape(NE, T)

    pos1 = jnp.sum(cumsum0(oh1) * oh1, axis=0, keepdims=True).astype(jnp.int32) - 1
    pos2 = jnp.sum(cumsum0(oh2) * oh2, axis=0, keepdims=True).astype(jnp.int32) - 1
    ok1 = pos1 < CAP
    ok2 = pos2 < CAP
    da1_ref[...] = jnp.where(ok1, e1 * CAP + pos1, SLOTS)
    da2_ref[...] = jnp.where(ok2, e2 * CAP + pos2, SLOTS)
    ga1_ref[...] = jnp.where(ok1, e1 * CAP + pos1, 0)
    ga2_ref[...] = jnp.where(ok2, e2 * CAP + pos2, 0)
    cw1_ref[...] = jnp.where(ok1, w1v, 0.0)
    cw2_ref[...] = jnp.where(ok2, w2v, 0.0)


_route = pl.pallas_call(
    _route_body,
    out_shape=[
        jax.ShapeDtypeStruct((T, D), jnp.float32),
        jax.ShapeDtypeStruct((1, T), jnp.int32),
        jax.ShapeDtypeStruct((1, T), jnp.int32),
        jax.ShapeDtypeStruct((1, T), jnp.int32),
        jax.ShapeDtypeStruct((1, T), jnp.int32),
        jax.ShapeDtypeStruct((1, T), jnp.float32),
        jax.ShapeDtypeStruct((1, T), jnp.float32),
    ],
)


@functools.cache
def _make_dispatch():
    mesh = plsc.VectorSubcoreMesh(core_axis_name="c", subcore_axis_name="s")

    @functools.partial(
        pl.kernel,
        mesh=mesh,
        out_type=jax.ShapeDtypeStruct((SLOTS, D), jnp.float32),
        scratch_types=[
            pltpu.VMEM((TPT, DC), jnp.float32),
            pltpu.VMEM((TPT,), jnp.int32),
            pltpu.VMEM((TPT,), jnp.int32),
            pltpu.VMEM_SHARED((BUFROWS, DC), jnp.float32),
            pltpu.SemaphoreType.DMA,
            pltpu.SemaphoreType.DMA,
            pltpu.SemaphoreType.DMA,
        ],
    )
    def _dispatch(xn_hbm, da1_hbm, da2_hbm, zeros_hbm, ei_hbm,
                  rows_v, idx1_v, idx2_v, shared, sem0, sem1, semr):
        c = lax.axis_index("c")
        s = lax.axis_index("s")
        base_t = s * TPT
        wbase = s * ORW
        pltpu.sync_copy(da1_hbm.at[pl.ds(base_t, TPT)], idx1_v)
        pltpu.sync_copy(da2_hbm.at[pl.ds(base_t, TPT)], idx2_v)
        pending = None
        for cj in range(NCH // 2):
            j = c * (NCH // 2) + cj
            if pending is not None:
                pending.wait()
            cp_z = pltpu.async_copy(zeros_hbm.at[pl.ds(wbase, ORW)],
                                    shared.at[pl.ds(wbase, ORW)], sem0)
            cp_l = pltpu.async_copy(
                xn_hbm.at[pl.ds(base_t, TPT), pl.ds(j * DC, DC)],
                rows_v, sem1)
            cp_z.wait()
            cp_l.wait()
            plsc.subcore_barrier()
            c1 = pltpu.async_copy(rows_v, shared.at[idx1_v], sem0, add=True)
            c2 = pltpu.async_copy(rows_v, shared.at[idx2_v], sem1, add=True)
            c1.wait()
            c2.wait()
            plsc.subcore_barrier()
            pending = pltpu.async_copy(
                shared.at[pl.ds(wbase, ORW)],
                ei_hbm.at[pl.ds(wbase, ORW), pl.ds(j * DC, DC)], semr)
        pending.wait()

    return _dispatch


def _ffn_body(ei_ref, w1_ref, w2_ref, w3_ref, out_ref):
    hb = pl.program_id(1)
    x = ei_ref[...]
    h1 = lax.dot_general(x, w1_ref[0], (((1,), (0,)), ((), ())),
                         preferred_element_type=jnp.float32)
    h2 = lax.dot_general(x, w2_ref[0], (((1,), (0,)), ((), ())),
                         preferred_element_type=jnp.float32)
    h = h1 * (1.0 / (1.0 + jnp.exp(-h1))) * h2
    part = lax.dot_general(h, w3_ref[0], (((1,), (0,)), ((), ())),
                           preferred_element_type=jnp.float32)

    @pl.when(hb == 0)
    def _():
        out_ref[...] = part

    @pl.when(hb != 0)
    def _():
        out_ref[...] += part


_ffn = pl.pallas_call(
    _ffn_body,
    grid=(NE, H // BH),
    in_specs=[
        pl.BlockSpec((CAP, D), lambda e, hb: (e, 0)),
        pl.BlockSpec((1, D, BH), lambda e, hb: (e, 0, hb)),
        pl.BlockSpec((1, D, BH), lambda e, hb: (e, 0, hb)),
        pl.BlockSpec((1, BH, D), lambda e, hb: (e, hb, 0)),
    ],
    out_specs=pl.BlockSpec((CAP, D), lambda e, hb: (e, 0)),
    out_shape=jax.ShapeDtypeStruct((SLOTS, D), jnp.float32),
)


@functools.cache
def _make_combine_gather():
    mesh = plsc.VectorSubcoreMesh(core_axis_name="c", subcore_axis_name="s")

    @functools.partial(
        pl.kernel,
        mesh=mesh,
        out_type=jax.ShapeDtypeStruct((2 * T, D), jnp.float32),
        scratch_types=[
            pltpu.VMEM((TPW,), jnp.int32),
            pltpu.VMEM((TPW, D), jnp.float32),
            pltpu.SemaphoreType.DMA,
        ],
    )
    def _combine_gather(eo_hbm, ga1_hbm, ga2_hbm, g_hbm, idx_v, rows_v, sem):
        c = lax.axis_index("c")
        s = lax.axis_index("s")
        wid = s * 2 + c
        base = wid * TPW
        pltpu.sync_copy(ga1_hbm.at[pl.ds(base, TPW)], idx_v)
        pltpu.async_copy(eo_hbm.at[idx_v], rows_v, sem).wait()
        pltpu.sync_copy(rows_v, g_hbm.at[pl.ds(base, TPW)])
        pltpu.sync_copy(ga2_hbm.at[pl.ds(base, TPW)], idx_v)
        pltpu.async_copy(eo_hbm.at[idx_v], rows_v, sem).wait()
        pltpu.sync_copy(rows_v, g_hbm.at[pl.ds(T + base, TPW)])

    return _combine_gather


def _wadd_body(g_ref, cw1_ref, cw2_ref, out_ref):
    out_ref[...] = cw1_ref[...] * g_ref[0] + cw2_ref[...] * g_ref[1]


_wadd = pl.pallas_call(
    _wadd_body,
    out_shape=jax.ShapeDtypeStruct((T, D), jnp.float32),
)


def kernel(x, norm_g, gate_w, w1, w2, w3):
    b, s, d = x.shape
    xf = x.reshape(T, D)
    xn, da1, da2, ga1, ga2, cw1, cw2 = _route(xf, norm_g.reshape(1, D), gate_w)
    zeros = jnp.zeros((SLOTS, DC), jnp.float32)
    ei = _make_dispatch()(xn, da1.reshape(T), da2.reshape(T), zeros)
    eo = _ffn(ei, w1, w2, w3)
    g = _make_combine_gather()(eo, ga1.reshape(T), ga2.reshape(T))
    out = _wadd(g.reshape(2, T, D), cw1.reshape(T, 1), cw2.reshape(T, 1))
    return out.reshape(b, s, d)
